# baseline (speedup 1.0000x reference)
import functools
import math

import jax
import jax.numpy as jnp
from jax import lax
from jax.experimental import pallas as pl
from jax.experimental.pallas import tpu as pltpu
from jax.experimental.pallas import tpu_sc as plsc

D_MODEL = 64
SCALE = math.sqrt(D_MODEL)
NUM_CORES = 2
NUM_SUBCORES = 16
NUM_WORKERS = NUM_CORES * NUM_SUBCORES
CHUNK = 128
NBUF = 3
LANES = 16
VB = 1024
NB_LO = 489
SPLIT = (NB_LO - 1) * VB
TAB_ROWS = NB_LO * VB


def _pack_block(lo_ref, hi_ref, out_ref):
    eye = jnp.eye(D_MODEL, dtype=jnp.float32) * SCALE
    dn = (((0,), (0,)), ((), ()))
    lo = lax.dot_general(lo_ref[...], eye, dn)
    hi = lax.dot_general(hi_ref[...], eye, dn)
    out_ref[...] = jnp.concatenate([lo, hi], axis=1)


def _pack_table(lutT):
    return pl.pallas_call(
        _pack_block,
        grid=(NB_LO,),
        in_specs=[
            pl.BlockSpec((D_MODEL, VB), lambda j: (0, j)),
            pl.BlockSpec((D_MODEL, VB), lambda j: (0, j + NB_LO - 1)),
        ],
        out_specs=pl.BlockSpec((VB, 2 * D_MODEL), lambda j: (j, 0)),
        out_shape=jax.ShapeDtypeStruct((TAB_ROWS, 2 * D_MODEL), jnp.float32),
    )(lutT, lutT)


@functools.partial(jax.jit, static_argnames=("batch",))
def _embed(xf, tab, batch):
    b_per_w = batch // NUM_WORKERS
    n_chunks = b_per_w // CHUNK
    n_main = n_chunks - 2
    assert n_main % NBUF == 0
    mesh = plsc.VectorSubcoreMesh(core_axis_name="c", subcore_axis_name="s")

    @functools.partial(
        pl.kernel,
        mesh=mesh,
        out_type=jax.ShapeDtypeStruct((batch, D_MODEL), jnp.float32),
        scratch_types=[
            pltpu.VMEM((NBUF, CHUNK), jnp.int32),
            pltpu.VMEM((NBUF, CHUNK), jnp.int32),
            pltpu.VMEM((NBUF, CHUNK, 2 * D_MODEL), jnp.float32),
            pltpu.VMEM((NBUF, CHUNK, D_MODEL), jnp.float32),
            [pltpu.SemaphoreType.DMA] * NBUF,
            [pltpu.SemaphoreType.DMA] * NBUF,
            [pltpu.SemaphoreType.DMA] * NBUF,
        ],
        compiler_params=pltpu.CompilerParams(use_tc_tiling_on_sc=True),
    )
    def emb(idx_hbm, tab_hbm, out_hbm, pidx_v, sidx_v, g_v, o_v, isem, gsem, ssem):
        wid = lax.axis_index("s") * NUM_CORES + lax.axis_index("c")
        base = wid * b_per_w

        def idx_slice(chunk):
            return idx_hbm.at[pl.ds(base + chunk * CHUNK, CHUNK)]

        def issue_idx(chunk, b):
            pltpu.async_copy(idx_slice(chunk), pidx_v.at[b], isem[b])

        def wait_idx(chunk, b):
            pltpu.make_async_copy(idx_slice(chunk), pidx_v.at[b], isem[b]).wait()

        def start_gather(chunk, b):
            for g in range(CHUNK // LANES):
                sl = pl.ds(g * LANES, LANES)
                v = pidx_v[b, sl]
                d = v - SPLIT
                neg = lax.shift_right_logical(d, 31)
                sidx_v[b, sl] = d + neg * SPLIT
            pltpu.async_copy(tab_hbm.at[sidx_v.at[b]], g_v.at[b], gsem[b])

        def wait_gather(b):
            pltpu.make_async_copy(
                tab_hbm.at[sidx_v.at[b]], g_v.at[b], gsem[b]
            ).wait()

        def select_scale(chunk, b):
            def grp_body(g, c):
                row0 = g * LANES
                idx16 = pidx_v[b, pl.ds(row0, LANES)]
                neg = lax.shift_right_logical(idx16 - SPLIT, 31)
                parf16 = (1 - neg).astype(jnp.float32)
                for l in range(LANES):
                    parf = lax.gather(
                        parf16,
                        jnp.full((LANES, 1), l, jnp.int32),
                        lax.GatherDimensionNumbers(
                            offset_dims=(),
                            collapsed_slice_dims=(0,),
                            start_index_map=(0,),
                        ),
                        (1,),
                        mode=lax.GatherScatterMode.PROMISE_IN_BOUNDS,
                    )
                    for t in range(D_MODEL // LANES):
                        sl = pl.ds(t * LANES, LANES)
                        lo = g_v[b, row0 + l, sl]
                        hi = g_v[b, row0 + l, pl.ds(D_MODEL + t * LANES, LANES)]
                        o_v[b, row0 + l, sl] = lo + (hi - lo) * parf
                return c

            lax.fori_loop(0, CHUNK // LANES, grp_body, 0)

        def out_slice(chunk):
            return out_hbm.at[pl.ds(base + chunk * CHUNK, CHUNK)]

        def start_store(chunk, b):
            pltpu.async_copy(o_v.at[b], out_slice(chunk), ssem[b])

        def wait_store(chunk, b):
            pltpu.make_async_copy(o_v.at[b], out_slice(chunk), ssem[b]).wait()

        pltpu.sync_copy(idx_slice(0), pidx_v.at[0])
        pltpu.sync_copy(idx_slice(1), pidx_v.at[1])
        start_gather(0, 0)
        start_gather(1, 1)

        def step(chunk, b):
            nb = (b + 2) % NBUF
            issue_idx(chunk + 2, nb)
            wait_gather(b)
            select_scale(chunk, b)
            start_store(chunk, b)
            wait_idx(chunk + 2, nb)

            @pl.when(chunk >= 1)
            def _():
                wait_store(chunk - 1, nb)

            start_gather(chunk + 2, nb)

        def main_body(s, c):
            for u in range(NBUF):
                step(s * NBUF + u, u)
            return c

        lax.fori_loop(0, n_main // NBUF, main_body, 0)

        for chunk in range(n_main, n_chunks):
            b = chunk % NBUF
            wait_gather(b)
            select_scale(chunk, b)
            start_store(chunk, b)

        for chunk in range(n_chunks - NBUF, n_chunks):
            wait_store(chunk, chunk % NBUF)

    return emb(xf, tab)


def kernel(x, lut):
    batch = x.size
    xf = x.reshape(batch).astype(jnp.int32)
    tab = _pack_table(lut.T)
    out = _embed(xf, tab, batch)
    return out.reshape(x.shape + (D_MODEL,))


# speedup vs baseline: 1.0007x; 1.0007x over previous
import functools
import math

import jax
import jax.numpy as jnp
from jax import lax
from jax.experimental import pallas as pl
from jax.experimental.pallas import tpu as pltpu
from jax.experimental.pallas import tpu_sc as plsc

D_MODEL = 64
SCALE = math.sqrt(D_MODEL)
NUM_CORES = 2
NUM_SUBCORES = 16
NUM_WORKERS = NUM_CORES * NUM_SUBCORES
CHUNK = 128
NBUF = 3
LANES = 16
VB = 2048
NB_LO = 245
SPLIT = (NB_LO - 1) * VB
TAB_ROWS = NB_LO * VB


def _pack_block(lo_ref, hi_ref, out_ref):
    eye = jnp.eye(D_MODEL, dtype=jnp.float32) * SCALE
    dn = (((0,), (0,)), ((), ()))
    lo = lax.dot_general(lo_ref[...], eye, dn)
    hi = lax.dot_general(hi_ref[...], eye, dn)
    out_ref[...] = jnp.concatenate([lo, hi], axis=1)


def _pack_table(lutT):
    return pl.pallas_call(
        _pack_block,
        grid=(NB_LO,),
        in_specs=[
            pl.BlockSpec((D_MODEL, VB), lambda j: (0, j)),
            pl.BlockSpec((D_MODEL, VB), lambda j: (0, j + NB_LO - 1)),
        ],
        out_specs=pl.BlockSpec((VB, 2 * D_MODEL), lambda j: (j, 0)),
        out_shape=jax.ShapeDtypeStruct((TAB_ROWS, 2 * D_MODEL), jnp.float32),
    )(lutT, lutT)


@functools.partial(jax.jit, static_argnames=("batch",))
def _embed(xf, tab, batch):
    b_per_w = batch // NUM_WORKERS
    n_chunks = b_per_w // CHUNK
    n_main = n_chunks - 2
    assert n_main % NBUF == 0
    mesh = plsc.VectorSubcoreMesh(core_axis_name="c", subcore_axis_name="s")

    @functools.partial(
        pl.kernel,
        mesh=mesh,
        out_type=jax.ShapeDtypeStruct((batch, D_MODEL), jnp.float32),
        scratch_types=[
            pltpu.VMEM((b_per_w,), jnp.int32),
            pltpu.VMEM((NBUF, CHUNK), jnp.int32),
            pltpu.VMEM((NBUF, CHUNK, 2 * D_MODEL), jnp.float32),
            pltpu.VMEM((NBUF, CHUNK, D_MODEL), jnp.float32),
            [pltpu.SemaphoreType.DMA] * NBUF,
            [pltpu.SemaphoreType.DMA] * NBUF,
        ],
        compiler_params=pltpu.CompilerParams(use_tc_tiling_on_sc=True),
    )
    def emb(idx_hbm, tab_hbm, out_hbm, idx_v, sidx_v, g_v, o_v, gsem, ssem):
        wid = lax.axis_index("s") * NUM_CORES + lax.axis_index("c")
        base = wid * b_per_w

        pltpu.sync_copy(idx_hbm.at[pl.ds(base, b_per_w)], idx_v)

        def start_gather(chunk, b):
            for g in range(CHUNK // LANES):
                sl = pl.ds(g * LANES, LANES)
                v = idx_v[pl.ds(chunk * CHUNK + g * LANES, LANES)]
                d = v - SPLIT
                neg = lax.shift_right_logical(d, 31)
                sidx_v[b, sl] = d + neg * SPLIT
            pltpu.async_copy(tab_hbm.at[sidx_v.at[b]], g_v.at[b], gsem[b])

        def wait_gather(b):
            pltpu.make_async_copy(
                tab_hbm.at[sidx_v.at[b]], g_v.at[b], gsem[b]
            ).wait()

        def select_scale(chunk, b):
            def grp_body(g, c):
                row0 = g * LANES
                idx16 = idx_v[pl.ds(chunk * CHUNK + row0, LANES)]
                neg = lax.shift_right_logical(idx16 - SPLIT, 31)
                parf16 = (1 - neg).astype(jnp.float32)

                def row_body(l, cc):
                    parf = lax.gather(
                        parf16,
                        jnp.full((LANES, 1), 0, jnp.int32) + l,
                        lax.GatherDimensionNumbers(
                            offset_dims=(),
                            collapsed_slice_dims=(0,),
                            start_index_map=(0,),
                        ),
                        (1,),
                        mode=lax.GatherScatterMode.PROMISE_IN_BOUNDS,
                    )
                    for t in range(D_MODEL // LANES):
                        sl = pl.ds(t * LANES, LANES)
                        lo = g_v[b, row0 + l, sl]
                        hi = g_v[b, row0 + l, pl.ds(D_MODEL + t * LANES, LANES)]
                        o_v[b, row0 + l, sl] = lo + (hi - lo) * parf
                    return cc

                lax.fori_loop(0, LANES, row_body, 0)
                return c

            lax.fori_loop(0, CHUNK // LANES, grp_body, 0)

        def out_slice(chunk):
            return out_hbm.at[pl.ds(base + chunk * CHUNK, CHUNK)]

        def start_store(chunk, b):
            pltpu.async_copy(o_v.at[b], out_slice(chunk), ssem[b])

        def wait_store(chunk, b):
            pltpu.make_async_copy(o_v.at[b], out_slice(chunk), ssem[b]).wait()

        start_gather(0, 0)
        start_gather(1, 1)

        def step(chunk, b):
            wait_gather(b)
            select_scale(chunk, b)
            start_store(chunk, b)
            nb = (b + 2) % NBUF

            @pl.when(chunk >= 1)
            def _():
                wait_store(chunk - 1, nb)

            start_gather(chunk + 2, nb)

        def main_body(s, c):
            for u in range(NBUF):
                step(s * NBUF + u, u)
            return c

        lax.fori_loop(0, n_main // NBUF, main_body, 0)

        for chunk in range(n_main, n_chunks):
            b = chunk % NBUF
            wait_gather(b)
            select_scale(chunk, b)
            start_store(chunk, b)

        for chunk in range(n_chunks - NBUF, n_chunks):
            wait_store(chunk, chunk % NBUF)

    return emb(xf, tab)


def kernel(x, lut):
    batch = x.size
    xf = x.reshape(batch).astype(jnp.int32)
    tab = _pack_table(lut.T)
    out = _embed(xf, tab, batch)
    return out.reshape(x.shape + (D_MODEL,))


# speedup vs baseline: 1.2447x; 1.2439x over previous
import functools
import math

import jax
import jax.numpy as jnp
from jax import lax
from jax.experimental import pallas as pl
from jax.experimental.pallas import tpu as pltpu
from jax.experimental.pallas import tpu_sc as plsc

D_MODEL = 64
SCALE = math.sqrt(D_MODEL)
NUM_CORES = 2
NUM_SUBCORES = 16
NUM_WORKERS = NUM_CORES * NUM_SUBCORES
CHUNK = 128
NBUF = 3
LANES = 16


@functools.partial(jax.jit, static_argnames=("batch",))
def _embed(xf, lut2, batch):
    b_per_w = batch // NUM_WORKERS
    n_chunks = b_per_w // CHUNK
    n_main = n_chunks - 2
    assert n_main % NBUF == 0
    mesh = plsc.VectorSubcoreMesh(core_axis_name="c", subcore_axis_name="s")

    @functools.partial(
        pl.kernel,
        mesh=mesh,
        out_type=jax.ShapeDtypeStruct((batch, D_MODEL), jnp.float32),
        scratch_types=[
            pltpu.VMEM((b_per_w,), jnp.int32),
            pltpu.VMEM((NBUF, CHUNK), jnp.int32),
            pltpu.VMEM((NBUF, CHUNK, 2 * D_MODEL), jnp.float32),
            pltpu.VMEM((NBUF, CHUNK, D_MODEL), jnp.float32),
            [pltpu.SemaphoreType.DMA] * NBUF,
            [pltpu.SemaphoreType.DMA] * NBUF,
        ],
        compiler_params=pltpu.CompilerParams(use_tc_tiling_on_sc=True),
    )
    def emb(idx_hbm, tab_hbm, out_hbm, idx_v, sidx_v, g_v, o_v, gsem, ssem):
        wid = lax.axis_index("s") * NUM_CORES + lax.axis_index("c")
        base = wid * b_per_w

        pltpu.sync_copy(idx_hbm.at[pl.ds(base, b_per_w)], idx_v)

        def start_gather(chunk, b):
            for g in range(CHUNK // LANES):
                sl = pl.ds(g * LANES, LANES)
                v = idx_v[pl.ds(chunk * CHUNK + g * LANES, LANES)]
                sidx_v[b, sl] = lax.shift_right_logical(v, 1)
            pltpu.async_copy(tab_hbm.at[sidx_v.at[b]], g_v.at[b], gsem[b])

        def wait_gather(b):
            pltpu.make_async_copy(
                tab_hbm.at[sidx_v.at[b]], g_v.at[b], gsem[b]
            ).wait()

        def select_scale(chunk, b):
            def grp_body(g, c):
                row0 = g * LANES
                idx16 = idx_v[pl.ds(chunk * CHUNK + row0, LANES)]
                parf16 = (idx16 & 1).astype(jnp.float32)
                for l in range(LANES):
                    parf = lax.gather(
                        parf16,
                        jnp.full((LANES, 1), l, jnp.int32),
                        lax.GatherDimensionNumbers(
                            offset_dims=(),
                            collapsed_slice_dims=(0,),
                            start_index_map=(0,),
                        ),
                        (1,),
                        mode=lax.GatherScatterMode.PROMISE_IN_BOUNDS,
                    )
                    for t in range(D_MODEL // LANES):
                        sl = pl.ds(t * LANES, LANES)
                        lo = g_v[b, row0 + l, sl]
                        hi = g_v[b, row0 + l, pl.ds(D_MODEL + t * LANES, LANES)]
                        o_v[b, row0 + l, sl] = (lo + (hi - lo) * parf) * SCALE
                return c

            lax.fori_loop(0, CHUNK // LANES, grp_body, 0)

        def out_slice(chunk):
            return out_hbm.at[pl.ds(base + chunk * CHUNK, CHUNK)]

        def start_store(chunk, b):
            pltpu.async_copy(o_v.at[b], out_slice(chunk), ssem[b])

        def wait_store(chunk, b):
            pltpu.make_async_copy(o_v.at[b], out_slice(chunk), ssem[b]).wait()

        start_gather(0, 0)
        start_gather(1, 1)

        def step(chunk, b):
            wait_gather(b)
            select_scale(chunk, b)
            start_store(chunk, b)
            nb = (b + 2) % NBUF

            @pl.when(chunk >= 1)
            def _():
                wait_store(chunk - 1, nb)

            start_gather(chunk + 2, nb)

        def main_body(s, c):
            for u in range(NBUF):
                step(s * NBUF + u, u)
            return c

        lax.fori_loop(0, n_main // NBUF, main_body, 0)

        for chunk in range(n_main, n_chunks):
            b = chunk % NBUF
            wait_gather(b)
            select_scale(chunk, b)
            start_store(chunk, b)

        for chunk in range(n_chunks - NBUF, n_chunks):
            wait_store(chunk, chunk % NBUF)

    return emb(xf, lut2)


def kernel(x, lut):
    batch = x.size
    xf = x.reshape(batch).astype(jnp.int32)
    lut2 = lut.reshape(lut.shape[0] // 2, 2 * D_MODEL)
    out = _embed(xf, lut2, batch)
    return out.reshape(x.shape + (D_MODEL,))
